# baseline (speedup 1.0000x reference)
import math

import jax
import jax.numpy as jnp
from jax.experimental import pallas as pl
from jax.experimental.pallas import tpu as pltpu

HIDDEN = 1024
EXPERTS = 64
K = 8
TOKENS = 32768
BLOCK_T = 4096
NBLK = TOKENS // BLOCK_T


def _router_block(x_ref, pk_ref, g_ref, w_ref, i_ref, scratch):
    i = pl.program_id(0)
    buf = jax.lax.rem(i, 2)

    @pl.when(i < NBLK)
    def _dot():
        logits = jax.lax.dot_general(
            pk_ref[...], x_ref[...],
            dimension_numbers=(((1,), (1,)), ((), ())),
            preferred_element_type=jnp.float32,
            precision=jax.lax.Precision.DEFAULT,
        )
        inv = 1.0 / math.sqrt(HIDDEN)
        scratch[buf] = jnp.maximum(logits * inv - g_ref[...], 0.0)

    @pl.when(i > 0)
    def _topk():
        vals = scratch[1 - buf]
        eidxf = jax.lax.broadcasted_iota(
            jnp.int32, (EXPERTS, BLOCK_T), 0).astype(jnp.float32)
        w_rows = []
        i_rows = []
        for k in range(K):
            m = jnp.max(vals, axis=0, keepdims=True)
            cand = jnp.where(vals == m, eidxf, float(EXPERTS))
            mi = jnp.min(cand, axis=0, keepdims=True)
            w_rows.append(m)
            i_rows.append(mi)
            if k < K - 1:
                vals = jnp.where(eidxf == mi, -1.0, vals)
        wT = jnp.concatenate(w_rows, axis=0)
        iT = jnp.concatenate(i_rows, axis=0).astype(jnp.int32)
        w_ref[...] = wT.T
        i_ref[...] = iT.T


@jax.jit
def kernel(x, proto_k, gate):
    gate2d = gate.reshape(EXPERTS, 1)
    w, idx = pl.pallas_call(
        _router_block,
        grid=(NBLK + 1,),
        in_specs=[
            pl.BlockSpec((BLOCK_T, HIDDEN), lambda i: (jnp.minimum(i, NBLK - 1), 0)),
            pl.BlockSpec((EXPERTS, HIDDEN), lambda i: (0, 0)),
            pl.BlockSpec((EXPERTS, 1), lambda i: (0, 0)),
        ],
        out_specs=[
            pl.BlockSpec((BLOCK_T, K), lambda i: (jnp.maximum(i - 1, 0), 0)),
            pl.BlockSpec((BLOCK_T, K), lambda i: (jnp.maximum(i - 1, 0), 0)),
        ],
        out_shape=[
            jax.ShapeDtypeStruct((TOKENS, K), jnp.float32),
            jax.ShapeDtypeStruct((TOKENS, K), jnp.int32),
        ],
        scratch_shapes=[pltpu.VMEM((2, EXPERTS, BLOCK_T), jnp.float32)],
    )(x, proto_k, gate2d)
    return (w, idx)


# speedup vs baseline: 1.0835x; 1.0835x over previous
import math

import jax
import jax.numpy as jnp
from jax.experimental import pallas as pl

HIDDEN = 1024
EXPERTS = 64
K = 8
TOKENS = 32768
BLOCK_T = 4096


def _router_block(x_ref, pk_ref, g_ref, w_ref, i_ref):
    logits = jax.lax.dot_general(
        pk_ref[...], x_ref[...],
        dimension_numbers=(((1,), (1,)), ((), ())),
        preferred_element_type=jnp.float32,
        precision=jax.lax.Precision.DEFAULT,
    )
    inv = 1.0 / math.sqrt(HIDDEN)
    vals = jnp.maximum(logits * inv - g_ref[...], 0.0)

    eidxf = jax.lax.broadcasted_iota(
        jnp.int32, (EXPERTS, BLOCK_T), 0).astype(jnp.float32)
    w_rows = []
    i_rows = []
    for k in range(K):
        m = jnp.max(vals, axis=0, keepdims=True)
        cand = jnp.where(vals == m, eidxf, float(EXPERTS))
        mi = jnp.min(cand, axis=0, keepdims=True)
        w_rows.append(m)
        i_rows.append(mi)
        if k < K - 1:
            vals = jnp.where(eidxf == mi, -1.0, vals)
    wT = jnp.concatenate(w_rows, axis=0)
    iT = jnp.concatenate(i_rows, axis=0).astype(jnp.int32)
    w_ref[...] = wT.T
    i_ref[...] = iT.T


@jax.jit
def kernel(x, proto_k, gate):
    gate2d = gate.reshape(EXPERTS, 1)
    grid = (TOKENS // BLOCK_T,)
    w, idx = pl.pallas_call(
        _router_block,
        grid=grid,
        in_specs=[
            pl.BlockSpec((BLOCK_T, HIDDEN), lambda i: (i, 0)),
            pl.BlockSpec((EXPERTS, HIDDEN), lambda i: (0, 0)),
            pl.BlockSpec((EXPERTS, 1), lambda i: (0, 0)),
        ],
        out_specs=[
            pl.BlockSpec((BLOCK_T, K), lambda i: (i, 0)),
            pl.BlockSpec((BLOCK_T, K), lambda i: (i, 0)),
        ],
        out_shape=[
            jax.ShapeDtypeStruct((TOKENS, K), jnp.float32),
            jax.ShapeDtypeStruct((TOKENS, K), jnp.int32),
        ],
    )(x, proto_k, gate2d)
    return (w, idx)
